# baseline (speedup 1.0000x reference)
import math

import jax
import jax.numpy as jnp
from jax import lax
from jax.experimental import pallas as pl
from jax.experimental.pallas import tpu as pltpu
from jax.experimental.pallas import tpu_sc as plsc

D_MODEL = 1024
SCALE = math.sqrt(D_MODEL)

NC = 2
NS = 16
NW = NC * NS
LANES = 16

B_TOTAL = 4 * 4096
B_PER_W = B_TOTAL // NW
CHUNK = 16
NCHUNK = B_PER_W // CHUNK
NGBUF = 4
NSBUF = 2


def _body(x_hbm, table_hbm, out_hbm, idx_v, gbufs, sbufs, gsems, ssems):
    wid = lax.axis_index("s") * NC + lax.axis_index("c")
    base = wid * B_PER_W

    pltpu.sync_copy(x_hbm.at[wid], idx_v)

    def start_gather(g):
        return pltpu.async_copy(
            table_hbm.at[idx_v.at[g]], gbufs[g % NGBUF], gsems[g % NGBUF])

    def start_scatter(g):
        return pltpu.async_copy(
            sbufs[g % NSBUF], out_hbm.at[pl.ds(base + g * CHUNK, CHUNK)],
            ssems[g % NSBUF])

    def compute(g):
        src = gbufs[g % NGBUF]
        dst = sbufs[g % NSBUF]

        @pl.loop(0, CHUNK)
        def _rows(r):
            @pl.loop(0, D_MODEL // LANES, unroll=8)
            def _cols(j):
                sl = (r, pl.ds(j * LANES, LANES))
                dst[sl] = src[sl] * SCALE

    gd = [None] * NCHUNK
    sd = [None] * NCHUNK
    for g in range(NGBUF - 1):
        gd[g] = start_gather(g)
    for g in range(NCHUNK):
        gd[g].wait()
        n = g + NGBUF - 1
        if n < NCHUNK:
            gd[n] = start_gather(n)
        if g >= NSBUF:
            sd[g - NSBUF].wait()
        compute(g)
        sd[g] = start_scatter(g)
    for g in range(NCHUNK - NSBUF, NCHUNK):
        sd[g].wait()


@jax.jit
def _emb_lookup(x_idx, table):
    mesh = plsc.VectorSubcoreMesh(core_axis_name="c", subcore_axis_name="s")
    run = pl.kernel(
        _body,
        out_type=jax.ShapeDtypeStruct((B_TOTAL, D_MODEL), jnp.float32),
        mesh=mesh,
        scratch_types=[
            pltpu.VMEM((NCHUNK, CHUNK), jnp.int32),
            tuple(pltpu.VMEM((CHUNK, D_MODEL), jnp.float32)
                  for _ in range(NGBUF)),
            tuple(pltpu.VMEM((CHUNK, D_MODEL), jnp.float32)
                  for _ in range(NSBUF)),
            tuple(pltpu.SemaphoreType.DMA for _ in range(NGBUF)),
            tuple(pltpu.SemaphoreType.DMA for _ in range(NSBUF)),
        ],
    )
    return run(x_idx, table)


def kernel(x, table):
    x_idx = x.reshape(NW, NCHUNK, CHUNK).astype(jnp.int32)
    out = _emb_lookup(x_idx, table)
    return out.reshape(x.shape + (D_MODEL,))


# speedup vs baseline: 2.6804x; 2.6804x over previous
import math

import jax
import jax.numpy as jnp
from jax import lax
from jax.experimental import pallas as pl
from jax.experimental.pallas import tpu as pltpu
from jax.experimental.pallas import tpu_sc as plsc

D_MODEL = 1024
SCALE = math.sqrt(D_MODEL)

NC = 2
NS = 16
NW = NC * NS
LANES = 16

B_TOTAL = 4 * 4096
B_PER_W = B_TOTAL // NW
CHUNK = 32
NCHUNK = B_PER_W // CHUNK
NBUF = 3


def _body(x_hbm, table_hbm, out_hbm, idx_v, bufs, gsems, ssems):
    wid = lax.axis_index("s") * NC + lax.axis_index("c")
    base = wid * B_PER_W

    pltpu.sync_copy(x_hbm.at[wid], idx_v)

    def start_gather(g):
        return pltpu.async_copy(
            table_hbm.at[idx_v.at[g]], bufs[g % NBUF], gsems[g % NBUF])

    def start_scatter(g):
        return pltpu.async_copy(
            bufs[g % NBUF], out_hbm.at[pl.ds(base + g * CHUNK, CHUNK)],
            ssems[g % NBUF])

    def compute(b):
        buf = bufs[b]

        @pl.loop(0, CHUNK)
        def _rows(r):
            @pl.loop(0, D_MODEL // LANES, unroll=8)
            def _cols(j):
                sl = (r, pl.ds(j * LANES, LANES))
                buf[sl] = buf[sl] * SCALE

    gd = [None] * NCHUNK
    sd = [None] * NCHUNK
    for g in range(NBUF - 1):
        gd[g] = start_gather(g)
    for g in range(NCHUNK):
        gd[g].wait()
        n = g + NBUF - 1
        if n < NCHUNK:
            if g >= 1:
                sd[g - 1].wait()
            gd[n] = start_gather(n)
        compute(g % NBUF)
        sd[g] = start_scatter(g)
    for g in range(NCHUNK - NBUF + 1, NCHUNK):
        sd[g].wait()


@jax.jit
def _emb_lookup(x_idx, table):
    mesh = plsc.VectorSubcoreMesh(core_axis_name="c", subcore_axis_name="s")
    run = pl.kernel(
        _body,
        out_type=jax.ShapeDtypeStruct((B_TOTAL, D_MODEL), jnp.float32),
        mesh=mesh,
        scratch_types=[
            pltpu.VMEM((NCHUNK, CHUNK), jnp.int32),
            tuple(pltpu.VMEM((CHUNK, D_MODEL), jnp.float32)
                  for _ in range(NBUF)),
            tuple(pltpu.SemaphoreType.DMA for _ in range(NBUF)),
            tuple(pltpu.SemaphoreType.DMA for _ in range(NBUF)),
        ],
    )
    return run(x_idx, table)


def kernel(x, table):
    x_idx = x.reshape(NW, NCHUNK, CHUNK).astype(jnp.int32)
    out = _emb_lookup(x_idx, table)
    return out.reshape(x.shape + (D_MODEL,))


# speedup vs baseline: 2.6884x; 1.0030x over previous
import math

import jax
import jax.numpy as jnp
from jax import lax
from jax.experimental import pallas as pl
from jax.experimental.pallas import tpu as pltpu
from jax.experimental.pallas import tpu_sc as plsc

D_MODEL = 1024
SCALE = math.sqrt(D_MODEL)

NC = 2
NS = 16
NW = NC * NS
LANES = 16

B_TOTAL = 4 * 4096
B_PER_W = B_TOTAL // NW
CHUNK = 32
NCHUNK = B_PER_W // CHUNK
NBUF = 3


def _body(x_hbm, table_hbm, out_hbm, idx_v, bufs, gsems, ssems):
    wid = lax.axis_index("s") * NC + lax.axis_index("c")
    base = wid * B_PER_W

    pltpu.sync_copy(x_hbm.at[wid, pl.ds(0, 8)], idx_v.at[pl.ds(0, 8)])

    def start_gather(g):
        return pltpu.async_copy(
            table_hbm.at[idx_v.at[g]], bufs[g % NBUF], gsems[g % NBUF])

    def start_scatter(g):
        return pltpu.async_copy(
            bufs[g % NBUF], out_hbm.at[pl.ds(base + g * CHUNK, CHUNK)],
            ssems[g % NBUF])

    def compute(b):
        buf = bufs[b]

        @pl.loop(0, CHUNK)
        def _rows(r):
            @pl.loop(0, D_MODEL // LANES, unroll=8)
            def _cols(j):
                sl = (r, pl.ds(j * LANES, LANES))
                buf[sl] = buf[sl] * SCALE

    gd = [None] * NCHUNK
    sd = [None] * NCHUNK
    for g in range(NBUF - 1):
        gd[g] = start_gather(g)
    pltpu.sync_copy(x_hbm.at[wid, pl.ds(8, NCHUNK - 8)],
                    idx_v.at[pl.ds(8, NCHUNK - 8)])
    for g in range(NCHUNK):
        gd[g].wait()
        n = g + NBUF - 1
        if n < NCHUNK:
            if g >= 1:
                sd[g - 1].wait()
            gd[n] = start_gather(n)
        compute(g % NBUF)
        sd[g] = start_scatter(g)
    for g in range(NCHUNK - NBUF + 1, NCHUNK):
        sd[g].wait()


@jax.jit
def _emb_lookup(x_idx, table):
    mesh = plsc.VectorSubcoreMesh(core_axis_name="c", subcore_axis_name="s")
    run = pl.kernel(
        _body,
        out_type=jax.ShapeDtypeStruct((B_TOTAL, D_MODEL), jnp.float32),
        mesh=mesh,
        scratch_types=[
            pltpu.VMEM((NCHUNK, CHUNK), jnp.int32),
            tuple(pltpu.VMEM((CHUNK, D_MODEL), jnp.float32)
                  for _ in range(NBUF)),
            tuple(pltpu.SemaphoreType.DMA for _ in range(NBUF)),
            tuple(pltpu.SemaphoreType.DMA for _ in range(NBUF)),
        ],
    )
    return run(x_idx, table)


def kernel(x, table):
    x_idx = x.reshape(NW, NCHUNK, CHUNK).astype(jnp.int32)
    out = _emb_lookup(x_idx, table)
    return out.reshape(x.shape + (D_MODEL,))
